# baseline (speedup 1.0000x reference)
import functools

import jax
import jax.numpy as jnp
from jax import lax
from jax.experimental import pallas as pl
from jax.experimental.pallas import tpu as pltpu
from jax.experimental.pallas import tpu_sc as plsc

N_NODES = 10000
N_EDGES = 320000
D = 128

NC = 2
NS = 16
NW = NC * NS
FPT = D // NW
COLW = FPT * N_NODES
NPAD = 10240
CHUNK = 3200
NCHUNKS = N_EDGES // CHUNK
NEG = -3.4028235e38

_mesh = plsc.VectorSubcoreMesh(
    core_axis_name="c", subcore_axis_name="s", num_cores=NC, num_subcores=NS
)


@functools.partial(
    pl.kernel,
    out_type=(
        jax.ShapeDtypeStruct((D * N_NODES,), jnp.float32),
        jax.ShapeDtypeStruct((NPAD,), jnp.float32),
    ),
    mesh=_mesh,
    compiler_params=pltpu.CompilerParams(needs_layout_passes=False),
    scratch_types=[
        pltpu.VMEM((COLW,), jnp.float32),
        pltpu.VMEM((COLW + 64,), jnp.float32),
        pltpu.VMEM((N_NODES + 16,), jnp.int32),
        pltpu.VMEM((NPAD,), jnp.float32),
        pltpu.VMEM((CHUNK,), jnp.int32),
        pltpu.VMEM((CHUNK,), jnp.int32),
        pltpu.VMEM((CHUNK,), jnp.int32),
        pltpu.VMEM((CHUNK,), jnp.int32),
        pltpu.SemaphoreType.DMA,
        pltpu.SemaphoreType.DMA,
        pltpu.SemaphoreType.DMA,
    ],
)
def _colmax_sc(src_hbm, dst_hbm, xt_hbm, segt_hbm, flag_hbm,
               xcol, accf, dsc, flgN, srcb0, srcb1, dstb0, dstb1,
               esem0, esem1, xsem):
    wid = lax.axis_index("s") * NC + lax.axis_index("c")
    base_col = wid * COLW

    srcb = (srcb0, srcb1)
    dstb = (dstb0, dstb1)
    esem = (esem0, esem1)

    neg = jnp.full((16,), NEG, jnp.float32)
    zero_f = jnp.zeros((16,), jnp.float32)
    one_f = jnp.ones((16,), jnp.float32)
    all_true = jnp.ones((16,), jnp.bool_)
    lane = lax.iota(jnp.int32, 16)

    hx = pltpu.async_copy(xt_hbm.at[pl.ds(base_col, COLW)], xcol, xsem)

    def issue_edges(c, p):
        base_e = c * CHUNK
        pltpu.async_copy(src_hbm.at[pl.ds(base_e, CHUNK)], srcb[p], esem[p])
        pltpu.async_copy(dst_hbm.at[pl.ds(base_e, CHUNK)], dstb[p], esem[p])

    def wait_edges(c, p):
        base_e = c * CHUNK
        pltpu.make_async_copy(src_hbm.at[pl.ds(base_e, CHUNK)], srcb[p], esem[p]).wait()
        pltpu.make_async_copy(dst_hbm.at[pl.ds(base_e, CHUNK)], dstb[p], esem[p]).wait()

    issue_edges(0, 0)
    issue_edges(1, 1)

    def init_acc(i, carry):
        for r in range(8):
            accf[pl.ds(i * 128 + r * 16, 16)] = neg
        return carry
    lax.fori_loop(0, (COLW + 127) // 128, init_acc, 0)

    def init_flg(i, carry):
        for r in range(8):
            flgN[pl.ds(i * 128 + r * 16, 16)] = zero_f
        return carry
    lax.fori_loop(0, NPAD // 128, init_flg, 0)

    hx.wait()

    def do_chunk(p):
        def group(i, carry):
            sv = srcb[p][pl.ds(i * 16, 16)]
            dv = dstb[p][pl.ds(i * 16, 16)]

            plsc.store_scatter(dsc, [dv], lane, mask=all_true)
            win = plsc.load_gather(dsc, [dv]) == lane

            @pl.when(wid == 0)
            def _():
                plsc.store_scatter(flgN, [dv], one_f, mask=all_true)

            for f in range(FPT):
                vals = plsc.load_gather(xcol, [sv + f * N_NODES])
                a = plsc.load_gather(accf, [dv + f * N_NODES])
                plsc.store_scatter(accf, [dv + f * N_NODES],
                                   jnp.maximum(a, vals), mask=win)

            def cond(rem):
                return jnp.any(rem)

            def body(rem):
                plsc.store_scatter(dsc, [dv], lane, mask=rem)
                w2 = (plsc.load_gather(dsc, [dv]) == lane) & rem
                for f in range(FPT):
                    vals = plsc.load_gather(xcol, [sv + f * N_NODES])
                    a = plsc.load_gather(accf, [dv + f * N_NODES])
                    plsc.store_scatter(accf, [dv + f * N_NODES],
                                       jnp.maximum(a, vals), mask=w2)
                return rem & (~w2)

            lax.while_loop(cond, body, ~win)
            return carry

        lax.fori_loop(0, CHUNK // 16, group, 0)

    def outer(i, carry):
        c0 = 2 * i
        wait_edges(c0, 0)
        do_chunk(0)

        @pl.when(i < (NCHUNKS // 2) - 1)
        def _():
            issue_edges(c0 + 2, 0)

        c1 = 2 * i + 1
        wait_edges(c1, 1)
        do_chunk(1)

        @pl.when(i < (NCHUNKS // 2) - 1)
        def _():
            issue_edges(c1 + 2, 1)

        return carry

    lax.fori_loop(0, NCHUNKS // 2, outer, 0)

    pltpu.sync_copy(accf.at[pl.ds(0, COLW)], segt_hbm.at[pl.ds(base_col, COLW)])

    @pl.when(wid == 0)
    def _():
        pltpu.sync_copy(flgN, flag_hbm)


_BR = 400


def _linear_body(x_ref, seg_ref, flag_ref, wt_ref, b_ref, o_ref):
    xb = x_ref[...]
    md = jnp.where(flag_ref[...] > 0.0, seg_ref[...] - xb, 0.0)
    cat = jnp.concatenate([xb, md], axis=1)
    o_ref[...] = (
        jnp.dot(cat, wt_ref[...], preferred_element_type=jnp.float32) + b_ref[...]
    )


@jax.jit
def _linear_tc(x, seg, flag, wt, b2):
    grid = N_NODES // _BR
    return pl.pallas_call(
        _linear_body,
        grid=(grid,),
        in_specs=[
            pl.BlockSpec((_BR, D), lambda i: (i, 0)),
            pl.BlockSpec((_BR, D), lambda i: (i, 0)),
            pl.BlockSpec((_BR, 1), lambda i: (i, 0)),
            pl.BlockSpec((2 * D, D), lambda i: (0, 0)),
            pl.BlockSpec((1, D), lambda i: (0, 0)),
        ],
        out_specs=pl.BlockSpec((_BR, D), lambda i: (i, 0)),
        out_shape=jax.ShapeDtypeStruct((N_NODES, D), jnp.float32),
    )(x, seg, flag, wt, b2)


def kernel(x, edge_index, W, b):
    ei = edge_index.astype(jnp.int32)
    src = ei[0]
    dst = ei[1]
    xt = x.T.reshape(-1)
    segt, flags = _colmax_sc(src, dst, xt)
    seg = segt.reshape(D, N_NODES).T
    flag = flags[:N_NODES].reshape(N_NODES, 1)
    wt = W.T
    return _linear_tc(x, seg, flag, wt, b.reshape(1, D))


# speedup vs baseline: 1.0683x; 1.0683x over previous
import functools

import jax
import jax.numpy as jnp
from jax import lax
from jax.experimental import pallas as pl
from jax.experimental.pallas import tpu as pltpu
from jax.experimental.pallas import tpu_sc as plsc

N_NODES = 10000
N_EDGES = 320000
D = 128

NC = 2
NS = 16
NW = NC * NS
FPT = D // NW
COLW = FPT * N_NODES
NPAD = 10240
CHUNK = 3200
NCHUNKS = N_EDGES // CHUNK
NEG = -3.4028235e38

_mesh = plsc.VectorSubcoreMesh(
    core_axis_name="c", subcore_axis_name="s", num_cores=NC, num_subcores=NS
)


@functools.partial(
    pl.kernel,
    out_type=(
        jax.ShapeDtypeStruct((D * N_NODES,), jnp.float32),
        jax.ShapeDtypeStruct((NPAD,), jnp.float32),
    ),
    mesh=_mesh,
    compiler_params=pltpu.CompilerParams(needs_layout_passes=False),
    scratch_types=[
        pltpu.VMEM((COLW,), jnp.float32),
        pltpu.VMEM((COLW + 64,), jnp.float32),
        pltpu.VMEM((N_NODES + 16,), jnp.int32),
        pltpu.VMEM((NPAD,), jnp.float32),
        pltpu.VMEM((CHUNK,), jnp.int32),
        pltpu.VMEM((CHUNK,), jnp.int32),
        pltpu.VMEM((CHUNK,), jnp.int32),
        pltpu.VMEM((CHUNK,), jnp.int32),
        pltpu.SemaphoreType.DMA,
        pltpu.SemaphoreType.DMA,
        pltpu.SemaphoreType.DMA,
    ],
)
def _colmax_sc(src_hbm, dst_hbm, xt_hbm, segt_hbm, flag_hbm,
               xcol, accf, dsc, flgN, srcb0, srcb1, dstb0, dstb1,
               esem0, esem1, xsem):
    wid = lax.axis_index("s") * NC + lax.axis_index("c")
    base_col = wid * COLW

    srcb = (srcb0, srcb1)
    dstb = (dstb0, dstb1)
    esem = (esem0, esem1)

    neg = jnp.full((16,), NEG, jnp.float32)
    zero_f = jnp.zeros((16,), jnp.float32)
    one_f = jnp.ones((16,), jnp.float32)
    all_true = jnp.ones((16,), jnp.bool_)
    lane = lax.iota(jnp.int32, 16)

    hx = pltpu.async_copy(xt_hbm.at[pl.ds(base_col, COLW)], xcol, xsem)

    def issue_edges(c, p):
        base_e = c * CHUNK
        pltpu.async_copy(src_hbm.at[pl.ds(base_e, CHUNK)], srcb[p], esem[p])
        pltpu.async_copy(dst_hbm.at[pl.ds(base_e, CHUNK)], dstb[p], esem[p])

    def wait_edges(c, p):
        base_e = c * CHUNK
        pltpu.make_async_copy(src_hbm.at[pl.ds(base_e, CHUNK)], srcb[p], esem[p]).wait()
        pltpu.make_async_copy(dst_hbm.at[pl.ds(base_e, CHUNK)], dstb[p], esem[p]).wait()

    issue_edges(0, 0)
    issue_edges(1, 1)

    def init_acc(i, carry):
        for r in range(8):
            accf[pl.ds(i * 128 + r * 16, 16)] = neg
        return carry
    lax.fori_loop(0, (COLW + 127) // 128, init_acc, 0)

    def init_flg(i, carry):
        for r in range(8):
            flgN[pl.ds(i * 128 + r * 16, 16)] = zero_f
        return carry
    lax.fori_loop(0, NPAD // 128, init_flg, 0)

    hx.wait()

    def do_chunk(p):
        def process16(off):
            sv = srcb[p][pl.ds(off, 16)]
            dv = dstb[p][pl.ds(off, 16)]

            plsc.store_scatter(dsc, [dv], lane, mask=all_true)
            win = plsc.load_gather(dsc, [dv]) == lane

            @pl.when(wid == 0)
            def _():
                plsc.store_scatter(flgN, [dv], one_f, mask=all_true)

            for f in range(FPT):
                vals = plsc.load_gather(xcol, [sv + f * N_NODES])
                a = plsc.load_gather(accf, [dv + f * N_NODES])
                plsc.store_scatter(accf, [dv + f * N_NODES],
                                   jnp.maximum(a, vals), mask=win)

            def cond(rem):
                return plsc.all_reduce_population_count(rem)[0] > 0

            def body(rem):
                plsc.store_scatter(dsc, [dv], lane, mask=rem)
                w2 = (plsc.load_gather(dsc, [dv]) == lane) & rem
                for f in range(FPT):
                    vals = plsc.load_gather(xcol, [sv + f * N_NODES])
                    a = plsc.load_gather(accf, [dv + f * N_NODES])
                    plsc.store_scatter(accf, [dv + f * N_NODES],
                                       jnp.maximum(a, vals), mask=w2)
                return rem & (~w2)

            lax.while_loop(cond, body, ~win)

        def group(i, carry):
            process16(i * 32)
            process16(i * 32 + 16)
            return carry

        lax.fori_loop(0, CHUNK // 32, group, 0)

    def outer(i, carry):
        c0 = 2 * i
        wait_edges(c0, 0)
        do_chunk(0)

        @pl.when(i < (NCHUNKS // 2) - 1)
        def _():
            issue_edges(c0 + 2, 0)

        c1 = 2 * i + 1
        wait_edges(c1, 1)
        do_chunk(1)

        @pl.when(i < (NCHUNKS // 2) - 1)
        def _():
            issue_edges(c1 + 2, 1)

        return carry

    lax.fori_loop(0, NCHUNKS // 2, outer, 0)

    pltpu.sync_copy(accf.at[pl.ds(0, COLW)], segt_hbm.at[pl.ds(base_col, COLW)])

    @pl.when(wid == 0)
    def _():
        pltpu.sync_copy(flgN, flag_hbm)


_BR = 400


def _linear_body(x_ref, seg_ref, flag_ref, wt_ref, b_ref, o_ref):
    xb = x_ref[...]
    md = jnp.where(flag_ref[...] > 0.0, seg_ref[...] - xb, 0.0)
    cat = jnp.concatenate([xb, md], axis=1)
    o_ref[...] = (
        jnp.dot(cat, wt_ref[...], preferred_element_type=jnp.float32) + b_ref[...]
    )


@jax.jit
def _linear_tc(x, seg, flag, wt, b2):
    grid = N_NODES // _BR
    return pl.pallas_call(
        _linear_body,
        grid=(grid,),
        in_specs=[
            pl.BlockSpec((_BR, D), lambda i: (i, 0)),
            pl.BlockSpec((_BR, D), lambda i: (i, 0)),
            pl.BlockSpec((_BR, 1), lambda i: (i, 0)),
            pl.BlockSpec((2 * D, D), lambda i: (0, 0)),
            pl.BlockSpec((1, D), lambda i: (0, 0)),
        ],
        out_specs=pl.BlockSpec((_BR, D), lambda i: (i, 0)),
        out_shape=jax.ShapeDtypeStruct((N_NODES, D), jnp.float32),
    )(x, seg, flag, wt, b2)


def kernel(x, edge_index, W, b):
    ei = edge_index.astype(jnp.int32)
    src = ei[0]
    dst = ei[1]
    xt = x.T.reshape(-1)
    segt, flags = _colmax_sc(src, dst, xt)
    seg = segt.reshape(D, N_NODES).T
    flag = flags[:N_NODES].reshape(N_NODES, 1)
    wt = W.T
    return _linear_tc(x, seg, flag, wt, b.reshape(1, D))
